# baseline (speedup 1.0000x reference)
import functools

import jax
import jax.numpy as jnp
from jax import lax
from jax.experimental import pallas as pl
from jax.experimental.pallas import tpu as pltpu
from jax.experimental.pallas import tpu_sc as plsc

B = 8
CIN = 3
H = 512
W = 512
PATCH = 32
GH = 16
GW = 16
CODE_DIM = 256
K = 8192
PD = CIN * PATCH * PATCH
N = B * GH * GW

ROW_BLK = 256
CB_BLK = 1024
N_ROW = N // ROW_BLK
N_CB = K // CB_BLK


def _vq_body(x_ref, we_ref, cb_ref, out_ref, cn_ref):
    i = pl.program_id(0)
    x5 = x_ref[0].reshape(CIN, GH, PATCH, GW, PATCH)
    p = x5.transpose(1, 3, 0, 2, 4).reshape(GH * GW, PD)
    z = jnp.dot(p, we_ref[...], preferred_element_type=jnp.float32)

    @pl.when(i == 0)
    def _norms():
        for j in range(N_CB):
            cbj = cb_ref[pl.ds(j * CB_BLK, CB_BLK), :]
            cn_ref[pl.ds(j * CB_BLK, CB_BLK), :] = jnp.sum(
                cbj * cbj, axis=1, keepdims=True)

    best_m = jnp.full((1, ROW_BLK), jnp.inf, dtype=jnp.float32)
    best_i = jnp.zeros((1, ROW_BLK), dtype=jnp.int32)
    for j in range(N_CB):
        cbj = cb_ref[pl.ds(j * CB_BLK, CB_BLK), :]
        st = lax.dot_general(cbj, z, (((1,), (1,)), ((), ())),
                             preferred_element_type=jnp.float32)
        dt = cn_ref[pl.ds(j * CB_BLK, CB_BLK), :] - 2.0 * st
        m = jnp.min(dt, axis=0, keepdims=True)
        ii = lax.broadcasted_iota(jnp.int32, dt.shape, 0)
        am = jnp.min(jnp.where(dt <= m, ii, K), axis=0, keepdims=True) + j * CB_BLK
        upd = m < best_m
        best_i = jnp.where(upd, am, best_i)
        best_m = jnp.where(upd, m, best_m)
    out_ref[...] = best_i.reshape(1, 1, ROW_BLK)


def _vq_indices(x, W_enc, codebook):
    nimg = x.shape[0]
    out = pl.pallas_call(
        _vq_body,
        grid=(nimg,),
        in_specs=[
            pl.BlockSpec((1, CIN, H, W), lambda i: (i, 0, 0, 0)),
            pl.BlockSpec((PD, CODE_DIM), lambda i: (0, 0)),
            pl.BlockSpec((K, CODE_DIM), lambda i: (0, 0)),
        ],
        out_specs=pl.BlockSpec((1, 1, ROW_BLK), lambda i: (i, 0, 0)),
        out_shape=jax.ShapeDtypeStruct((nimg, 1, ROW_BLK), jnp.int32),
        scratch_shapes=[
            pltpu.VMEM((K, 1), jnp.float32),
        ],
        compiler_params=pltpu.CompilerParams(
            dimension_semantics=("arbitrary",),
        ),
    )(x, W_enc, codebook)
    return out.reshape(nimg * ROW_BLK)


NW = 32


def _sc_gather_body(n_rows, cb_hbm, idx_hbm, out_hbm, idx_v, rows_v, sem):
    b_per_w = n_rows // NW
    wid = lax.axis_index("s") * 2 + lax.axis_index("c")
    base = wid * b_per_w
    pltpu.sync_copy(idx_hbm.at[pl.ds(base, b_per_w)], idx_v)
    pltpu.async_copy(cb_hbm.at[idx_v], rows_v, sem).wait()
    pltpu.sync_copy(rows_v, out_hbm.at[pl.ds(base, b_per_w)])


def _sc_gather(codebook, inds):
    n_rows = inds.shape[0]
    b_per_w = n_rows // NW
    k = functools.partial(
        pl.kernel,
        out_type=jax.ShapeDtypeStruct((n_rows, CODE_DIM), jnp.float32),
        mesh=plsc.VectorSubcoreMesh(core_axis_name="c", subcore_axis_name="s"),
        scratch_types=[
            pltpu.VMEM((b_per_w,), jnp.int32),
            pltpu.VMEM((b_per_w, CODE_DIM), jnp.float32),
            pltpu.SemaphoreType.DMA,
        ],
    )(functools.partial(_sc_gather_body, n_rows))
    return k(codebook, inds)


def _dec_body(hq_ref, wd_ref, out_ref):
    op = jnp.dot(hq_ref[...], wd_ref[...],
                 preferred_element_type=jnp.float32)
    op5 = op.reshape(GH, GW, CIN, PATCH, PATCH)
    out_ref[...] = op5.transpose(2, 0, 3, 1, 4).reshape(1, CIN, H, W)


def _decode(hq, W_dec):
    return pl.pallas_call(
        _dec_body,
        grid=(N_ROW,),
        in_specs=[
            pl.BlockSpec((ROW_BLK, CODE_DIM), lambda i: (i, 0)),
            pl.BlockSpec((CODE_DIM, PD), lambda i: (0, 0)),
        ],
        out_specs=pl.BlockSpec((1, CIN, H, W), lambda i: (i, 0, 0, 0)),
        out_shape=jax.ShapeDtypeStruct((B, CIN, H, W), jnp.float32),
        compiler_params=pltpu.CompilerParams(
            dimension_semantics=("parallel",),
        ),
    )(hq, W_dec)


def kernel(x, W_enc, codebook, W_dec):
    i0 = _vq_indices(x[: B // 2], W_enc, codebook)
    g0 = _sc_gather(codebook, i0)
    i1 = _vq_indices(x[B // 2:], W_enc, codebook)
    g1 = _sc_gather(codebook, i1)
    hq = jnp.concatenate([g0, g1], axis=0)
    return _decode(hq, W_dec)


# speedup vs baseline: 1.1932x; 1.1932x over previous
import functools

import jax
import jax.numpy as jnp
from jax import lax
from jax.experimental import pallas as pl
from jax.experimental.pallas import tpu as pltpu
from jax.experimental.pallas import tpu_sc as plsc

B = 8
CIN = 3
H = 512
W = 512
PATCH = 32
GH = 16
GW = 16
CODE_DIM = 256
K = 8192
PD = CIN * PATCH * PATCH
N = B * GH * GW

ROW_BLK = 256
CB_BLK = 1024
N_ROW = N // ROW_BLK
N_CB = K // CB_BLK


def _vq_body(x_ref, we_ref, cb_ref, out_ref, cn_ref):
    i = pl.program_id(0)
    x5 = x_ref[0].reshape(CIN, GH, PATCH, GW, PATCH)
    p = x5.transpose(1, 3, 0, 2, 4).reshape(GH * GW, PD)
    z = jnp.dot(p, we_ref[...], preferred_element_type=jnp.float32)

    @pl.when(i == 0)
    def _norms():
        for j in range(N_CB):
            cbj = cb_ref[pl.ds(j * CB_BLK, CB_BLK), :]
            cn_ref[pl.ds(j * CB_BLK, CB_BLK), :] = jnp.sum(
                cbj * cbj, axis=1, keepdims=True)

    best_m = jnp.full((1, ROW_BLK), jnp.inf, dtype=jnp.float32)
    best_i = jnp.zeros((1, ROW_BLK), dtype=jnp.int32)
    for j in range(N_CB):
        cbj = cb_ref[pl.ds(j * CB_BLK, CB_BLK), :]
        st = lax.dot_general(cbj, z, (((1,), (1,)), ((), ())),
                             preferred_element_type=jnp.float32)
        dt = cn_ref[pl.ds(j * CB_BLK, CB_BLK), :] - 2.0 * st
        m = jnp.min(dt, axis=0, keepdims=True)
        ii = lax.broadcasted_iota(jnp.int32, dt.shape, 0)
        am = jnp.min(jnp.where(dt <= m, ii, K), axis=0, keepdims=True) + j * CB_BLK
        upd = m < best_m
        best_i = jnp.where(upd, am, best_i)
        best_m = jnp.where(upd, m, best_m)
    out_ref[...] = best_i.reshape(1, 1, ROW_BLK)


def _vq_indices(x, W_enc, codebook):
    nimg = x.shape[0]
    out = pl.pallas_call(
        _vq_body,
        grid=(nimg,),
        in_specs=[
            pl.BlockSpec((1, CIN, H, W), lambda i: (i, 0, 0, 0)),
            pl.BlockSpec((PD, CODE_DIM), lambda i: (0, 0)),
            pl.BlockSpec((K, CODE_DIM), lambda i: (0, 0)),
        ],
        out_specs=pl.BlockSpec((1, 1, ROW_BLK), lambda i: (i, 0, 0)),
        out_shape=jax.ShapeDtypeStruct((nimg, 1, ROW_BLK), jnp.int32),
        scratch_shapes=[
            pltpu.VMEM((K, 1), jnp.float32),
        ],
        compiler_params=pltpu.CompilerParams(
            dimension_semantics=("arbitrary",),
        ),
    )(x, W_enc, codebook)
    return out.reshape(nimg * ROW_BLK)


NW = 32


def _sc_gather_body(n_rows, cb_hbm, idx_hbm, out_hbm, idx_v, rows_v, sem):
    b_per_w = n_rows // NW
    wid = lax.axis_index("s") * 2 + lax.axis_index("c")
    base = wid * b_per_w
    pltpu.sync_copy(idx_hbm.at[pl.ds(base, b_per_w)], idx_v)
    pltpu.async_copy(cb_hbm.at[idx_v], rows_v, sem).wait()
    pltpu.sync_copy(rows_v, out_hbm.at[pl.ds(base, b_per_w)])


def _sc_gather(codebook, inds):
    n_rows = inds.shape[0]
    b_per_w = n_rows // NW
    k = functools.partial(
        pl.kernel,
        out_type=jax.ShapeDtypeStruct((n_rows, CODE_DIM), jnp.float32),
        mesh=plsc.VectorSubcoreMesh(core_axis_name="c", subcore_axis_name="s"),
        scratch_types=[
            pltpu.VMEM((b_per_w,), jnp.int32),
            pltpu.VMEM((b_per_w, CODE_DIM), jnp.float32),
            pltpu.SemaphoreType.DMA,
        ],
    )(functools.partial(_sc_gather_body, n_rows))
    return k(codebook, inds)


def _dec_body(hq_ref, wd_ref, out_ref):
    op = jnp.dot(hq_ref[...], wd_ref[...],
                 preferred_element_type=jnp.float32).astype(jnp.bfloat16)
    op5 = op.reshape(GH, GW, CIN, PATCH, PATCH)
    t = op5.transpose(2, 0, 3, 1, 4).reshape(1, CIN, H, W)
    out_ref[...] = t.astype(jnp.float32)


def _decode(hq, W_dec):
    return pl.pallas_call(
        _dec_body,
        grid=(N_ROW,),
        in_specs=[
            pl.BlockSpec((ROW_BLK, CODE_DIM), lambda i: (i, 0)),
            pl.BlockSpec((CODE_DIM, PD), lambda i: (0, 0)),
        ],
        out_specs=pl.BlockSpec((1, CIN, H, W), lambda i: (i, 0, 0, 0)),
        out_shape=jax.ShapeDtypeStruct((B, CIN, H, W), jnp.float32),
        compiler_params=pltpu.CompilerParams(
            dimension_semantics=("parallel",),
        ),
    )(hq, W_dec)


def kernel(x, W_enc, codebook, W_dec):
    inds = _vq_indices(x, W_enc, codebook)
    hq = _sc_gather(codebook, inds)
    return _decode(hq.astype(jnp.bfloat16), W_dec.astype(jnp.bfloat16))
